# baseline (speedup 1.0000x reference)
import functools

import jax
import jax.numpy as jnp
from jax import lax
from jax.experimental import pallas as pl
from jax.experimental.pallas import tpu as pltpu
from jax.experimental.pallas import tpu_sc as plsc

_INPUT_DIM = 2048
_NUM_EXPERTS = 64
_N_TOKENS = 16384
_TBLK = 2048

_NW = 32
_TPW = _N_TOKENS // _NW
_LANES = 16


def _logits_kernel(x_ref, w_ref, out_ref):
    out_ref[...] = jax.lax.dot_general(
        w_ref[...], x_ref[...],
        dimension_numbers=(((1,), (1,)), ((), ())),
        preferred_element_type=jnp.float32,
    )


def _logits_t(x, W):
    return pl.pallas_call(
        _logits_kernel,
        grid=(_N_TOKENS // _TBLK,),
        in_specs=[
            pl.BlockSpec((_TBLK, _INPUT_DIM), lambda i: (i, 0)),
            pl.BlockSpec((_NUM_EXPERTS, _INPUT_DIM), lambda i: (0, 0)),
        ],
        out_specs=pl.BlockSpec((_NUM_EXPERTS, _TBLK), lambda i: (0, i)),
        out_shape=jax.ShapeDtypeStruct((_NUM_EXPERTS, _N_TOKENS), jnp.float32),
        compiler_params=pltpu.CompilerParams(
            dimension_semantics=("arbitrary",),
        ),
    )(x, W)


@functools.partial(
    pl.kernel,
    out_type=[
        jax.ShapeDtypeStruct((_N_TOKENS,), jnp.int32),
        jax.ShapeDtypeStruct((_N_TOKENS,), jnp.int32),
        jax.ShapeDtypeStruct((_N_TOKENS,), jnp.float32),
        jax.ShapeDtypeStruct((_N_TOKENS,), jnp.float32),
    ],
    mesh=plsc.VectorSubcoreMesh(core_axis_name="c", subcore_axis_name="s"),
    scratch_types=[
        pltpu.VMEM((_NUM_EXPERTS, _TPW), jnp.float32),
        pltpu.VMEM((_TPW,), jnp.int32),
        pltpu.VMEM((_TPW,), jnp.int32),
        pltpu.VMEM((_TPW,), jnp.float32),
        pltpu.VMEM((_TPW,), jnp.float32),
    ],
)
def _sc_top2(logits_hbm, i1_hbm, i2_hbm, v1_hbm, v2_hbm,
             slab, i1_v, i2_v, v1_v, v2_v):
    wid = lax.axis_index("s") * 2 + lax.axis_index("c")
    base = wid * _TPW
    pltpu.sync_copy(logits_hbm.at[:, pl.ds(base, _TPW)], slab)

    def chunk_body(c, carry):
        t0 = c * _LANES
        m1 = slab[0, pl.ds(t0, _LANES)]
        i1 = jnp.zeros((_LANES,), jnp.int32)
        m2 = jnp.full((_LANES,), -jnp.inf, jnp.float32)
        i2 = jnp.zeros((_LANES,), jnp.int32)
        for e in range(1, _NUM_EXPERTS):
            l = slab[e, pl.ds(t0, _LANES)]
            gt1 = l > m1
            gt2 = l > m2
            ei = jnp.full((_LANES,), e, jnp.int32)
            i2 = jnp.where(gt1, i1, jnp.where(gt2, ei, i2))
            m2 = jnp.where(gt1, m1, jnp.where(gt2, l, m2))
            i1 = jnp.where(gt1, ei, i1)
            m1 = jnp.where(gt1, l, m1)
        e2 = jnp.exp(m2 - m1)
        s = 1.0 + e2
        i1_v[pl.ds(t0, _LANES)] = i1
        i2_v[pl.ds(t0, _LANES)] = i2
        v1_v[pl.ds(t0, _LANES)] = 1.0 / s
        v2_v[pl.ds(t0, _LANES)] = e2 / s
        return carry

    lax.fori_loop(0, _TPW // _LANES, chunk_body, 0)
    pltpu.sync_copy(i1_v, i1_hbm.at[pl.ds(base, _TPW)])
    pltpu.sync_copy(i2_v, i2_hbm.at[pl.ds(base, _TPW)])
    pltpu.sync_copy(v1_v, v1_hbm.at[pl.ds(base, _TPW)])
    pltpu.sync_copy(v2_v, v2_hbm.at[pl.ds(base, _TPW)])


def kernel(x, W):
    logits_t = _logits_t(x, W)
    i1, i2, v1, v2 = _sc_top2(logits_t)
    idx = jnp.concatenate([i1[:, None], i2[:, None]], axis=1)
    val = jnp.concatenate([v1[:, None], v2[:, None]], axis=1)
    return (idx, val)
